# baseline (speedup 1.0000x reference)
import jax
import jax.numpy as jnp
from jax import lax
from jax.experimental import pallas as pl
from jax.experimental.pallas import tpu as pltpu
from jax.experimental.pallas import tpu_sc as plsc

N = 50000
H = 16
K = 15
SIGMA = 0.05

_NC = 2
_NS = 16
_NW = _NC * _NS
_ROWS = N * H
_G = 128
_GPW = 196
_ROWS_PAD = _NW * _GPW * _G
_GPC = 14
_NCH = _GPW // _GPC
_CHR = _GPC * _G

_NPTS = _ROWS_PAD // H
_PPW = _NPTS // _NW
_PPC = _CHR // H

_NPB = 128
_TCB = 256
_KP = 16


def _sc_gather(tbl_hbm, idx_hbm, out_hbm,
               idx_v, rows_v, wide_v, ident_v, qrows_v, sem):
    wid = lax.axis_index("s") * _NC + lax.axis_index("c")
    iota = lax.iota(jnp.int32, 16)
    m4 = iota < 4
    m8 = iota < 8
    m12 = iota < 12
    gbase = wid * _GPW
    pbase = wid * _PPW

    def body(i, carry):
        g0 = gbase + i * _GPC
        p0 = pbase + i * _PPC
        pltpu.sync_copy(idx_hbm.at[pl.ds(g0, _GPC), :], idx_v)
        for g in range(_PPC // 16):
            ident_v[pl.ds(g * 16, 16)] = p0 + g * 16 + iota
        for j in range(_GPC):
            pltpu.async_copy(
                tbl_hbm.at[idx_v.at[j]],
                rows_v.at[pl.ds(j * _G, _G), :],
                sem,
            )
        pltpu.async_copy(tbl_hbm.at[ident_v], qrows_v, sem)
        for j in range(_GPC):
            pltpu.make_async_copy(
                tbl_hbm.at[idx_v.at[j]],
                rows_v.at[pl.ds(j * _G, _G), :],
                sem,
            ).wait()
        pltpu.make_async_copy(tbl_hbm.at[ident_v], qrows_v, sem).wait()

        def rp(p, c2):
            m0 = p * 16
            for s in range(4):
                v = jnp.where(
                    m4, rows_v[m0 + 4 * s, :],
                    jnp.where(m8, rows_v[m0 + 4 * s + 1, :],
                              jnp.where(m12, rows_v[m0 + 4 * s + 2, :],
                                        rows_v[m0 + 4 * s + 3, :])))
                wide_v[p, pl.ds(s * 16, 16)] = v
            wide_v[p, pl.ds(64, 16)] = qrows_v[p, :]
            return c2

        lax.fori_loop(0, _PPC, rp, 0)
        pltpu.sync_copy(wide_v, out_hbm.at[pl.ds(p0, _PPC), :])
        return carry

    lax.fori_loop(0, _NCH, body, 0)


def _half(gt, kp_ref, w2t_ref):
    kp = kp_ref[...]
    kpx = kp[:, 0:1]
    kpy = kp[:, 1:2]
    kpz = kp[:, 2:3]
    xq = gt[64:65, :]
    yq = gt[65:66, :]
    zq = gt[66:67, :]

    wfx = jnp.zeros((_KP, _NPB), jnp.float32)
    wfy = jnp.zeros((_KP, _NPB), jnp.float32)
    wfz = jnp.zeros((_KP, _NPB), jnp.float32)
    cnt = jnp.zeros((1, _NPB), jnp.float32)
    for h in range(H):
        xh = gt[4 * h:4 * h + 1, :]
        yh = gt[4 * h + 1:4 * h + 2, :]
        zh = gt[4 * h + 2:4 * h + 3, :]
        dx = (xh - xq) - kpx
        dy = (yh - yq) - kpy
        dz = (zh - zq) - kpz
        sq = dx * dx + dy * dy + dz * dz
        w = jnp.maximum(1.0 - jnp.sqrt(sq) * (1.0 / SIGMA), 0.0)
        wfx = wfx + w * xh
        wfy = wfy + w * yh
        wfz = wfz + w * zh
        cnt = cnt + (xh + yh + zh > 0.0).astype(jnp.float32)

    g = jnp.concatenate([wfx, wfy, wfz], axis=0)
    outT = jnp.dot(w2t_ref[...], g,
                   preferred_element_type=jnp.float32)
    recip = 1.0 / jnp.maximum(cnt, 1.0)
    return outT * recip


def _tc_body(gath_ref, kp_ref, w2t_ref, bias_ref, out_ref):
    for s in range(_TCB // _NPB):
        gt = gath_ref[pl.ds(s * _NPB, _NPB), :].T
        outT = _half(gt, kp_ref, w2t_ref) + bias_ref[...]
        out_ref[pl.ds(s * _NPB, _NPB), :] = outT.T


def kernel(points, neighbor_indices, weights, bias, kernel_points):
    idx32 = neighbor_indices.reshape(-1).astype(jnp.int32)
    idx2d = jnp.pad(idx32, (0, _ROWS_PAD - _ROWS)).reshape(-1, _G)
    tbl4 = jnp.pad(points, ((0, _NPTS - N), (0, 1)))
    tbl16 = jnp.tile(tbl4, (1, 4))
    kp_pad = jnp.concatenate(
        [kernel_points, jnp.full((1, 3), 1e4, jnp.float32)], axis=0)
    w2 = jnp.pad(jnp.transpose(weights, (1, 0, 2)),
                 ((0, 0), (0, 1), (0, 0)))
    w2t = w2.reshape(48, 32).T
    bias2 = bias.reshape(32, 1)

    mesh = plsc.VectorSubcoreMesh(core_axis_name="c", subcore_axis_name="s")
    gathered = pl.kernel(
        _sc_gather,
        out_type=jax.ShapeDtypeStruct((_NPTS, _G), jnp.float32),
        mesh=mesh,
        scratch_types=[
            pltpu.VMEM((_GPC, _G), jnp.int32),
            pltpu.VMEM((_CHR, 16), jnp.float32),
            pltpu.VMEM((_PPC, _G), jnp.float32),
            pltpu.VMEM((_PPC,), jnp.int32),
            pltpu.VMEM((_PPC, 16), jnp.float32),
            pltpu.SemaphoreType.DMA,
        ],
        compiler_params=pltpu.CompilerParams(use_tc_tiling_on_sc=False),
    )(tbl16, idx2d)

    grid = (N + _TCB - 1) // _TCB
    out = pl.pallas_call(
        _tc_body,
        grid=(grid,),
        in_specs=[
            pl.BlockSpec((_TCB, _G), lambda i: (i, 0)),
            pl.BlockSpec((_KP, 3), lambda i: (0, 0)),
            pl.BlockSpec((32, 48), lambda i: (0, 0)),
            pl.BlockSpec((32, 1), lambda i: (0, 0)),
        ],
        out_specs=pl.BlockSpec((_TCB, 32), lambda i: (i, 0)),
        out_shape=jax.ShapeDtypeStruct((N, 32), jnp.float32),
    )(gathered, kp_pad, w2t, bias2)
    return out


# speedup vs baseline: 1.1023x; 1.1023x over previous
import jax
import jax.numpy as jnp
from jax import lax
from jax.experimental import pallas as pl
from jax.experimental.pallas import tpu as pltpu
from jax.experimental.pallas import tpu_sc as plsc

N = 50000
H = 16
K = 15
SIGMA = 0.05

_NC = 2
_NS = 16
_NW = _NC * _NS
_ROWS = N * H
_G = 128
_GPW = 196
_ROWS_PAD = _NW * _GPW * _G
_GPC = 14
_NCH = _GPW // _GPC
_CHR = _GPC * _G

_NPTS = _ROWS_PAD // H
_PPW = _NPTS // _NW
_PPC = _CHR // H

_NPB = 128
_TCB = 256
_KP = 16


def _sc_gather(tbl_hbm, idx_hbm, out_hbm,
               idx_a, rows_a, ident_a, qrows_a, sem_a,
               idx_b, rows_b, ident_b, qrows_b, sem_b,
               wide_v):
    wid = lax.axis_index("s") * _NC + lax.axis_index("c")
    iota = lax.iota(jnp.int32, 16)
    m4 = iota < 4
    m8 = iota < 8
    m12 = iota < 12
    gbase = wid * _GPW
    pbase = wid * _PPW
    bufs = [(idx_a, rows_a, ident_a, qrows_a, sem_a),
            (idx_b, rows_b, ident_b, qrows_b, sem_b)]

    def fire(i, buf):
        idx_v, rows_v, ident_v, qrows_v, sem = buf
        pltpu.sync_copy(idx_hbm.at[pl.ds(gbase + i * _GPC, _GPC), :], idx_v)
        p0 = pbase + i * _PPC
        for g in range(_PPC // 16):
            ident_v[pl.ds(g * 16, 16)] = p0 + g * 16 + iota
        for j in range(_GPC):
            pltpu.async_copy(
                tbl_hbm.at[idx_v.at[j]],
                rows_v.at[pl.ds(j * _G, _G), :],
                sem,
            )
        pltpu.async_copy(tbl_hbm.at[ident_v], qrows_v, sem)

    def drain_and_emit(i, buf):
        idx_v, rows_v, ident_v, qrows_v, sem = buf
        for j in range(_GPC):
            pltpu.make_async_copy(
                tbl_hbm.at[idx_v.at[j]],
                rows_v.at[pl.ds(j * _G, _G), :],
                sem,
            ).wait()
        pltpu.make_async_copy(tbl_hbm.at[ident_v], qrows_v, sem).wait()

        def rp(p, c2):
            m0 = p * 16
            for s in range(4):
                v = jnp.where(
                    m4, rows_v[m0 + 4 * s, :],
                    jnp.where(m8, rows_v[m0 + 4 * s + 1, :],
                              jnp.where(m12, rows_v[m0 + 4 * s + 2, :],
                                        rows_v[m0 + 4 * s + 3, :])))
                wide_v[p, pl.ds(s * 16, 16)] = v
            wide_v[p, pl.ds(64, 16)] = qrows_v[p, :]
            return c2

        lax.fori_loop(0, _PPC, rp, 0)
        pltpu.sync_copy(
            wide_v, out_hbm.at[pl.ds(pbase + i * _PPC, _PPC), :])

    fire(0, bufs[0])
    for i in range(_NCH):
        if i + 1 < _NCH:
            fire(i + 1, bufs[(i + 1) % 2])
        drain_and_emit(i, bufs[i % 2])


def _half(gt, kp_ref, w2t_ref):
    kp = kp_ref[...]
    kpx = kp[:, 0:1]
    kpy = kp[:, 1:2]
    kpz = kp[:, 2:3]
    xq = gt[64:65, :]
    yq = gt[65:66, :]
    zq = gt[66:67, :]

    wfx = jnp.zeros((_KP, _NPB), jnp.float32)
    wfy = jnp.zeros((_KP, _NPB), jnp.float32)
    wfz = jnp.zeros((_KP, _NPB), jnp.float32)
    cnt = jnp.zeros((1, _NPB), jnp.float32)
    for h in range(H):
        xh = gt[4 * h:4 * h + 1, :]
        yh = gt[4 * h + 1:4 * h + 2, :]
        zh = gt[4 * h + 2:4 * h + 3, :]
        dx = (xh - xq) - kpx
        dy = (yh - yq) - kpy
        dz = (zh - zq) - kpz
        sq = dx * dx + dy * dy + dz * dz
        w = jnp.maximum(1.0 - jnp.sqrt(sq) * (1.0 / SIGMA), 0.0)
        wfx = wfx + w * xh
        wfy = wfy + w * yh
        wfz = wfz + w * zh
        cnt = cnt + (xh + yh + zh > 0.0).astype(jnp.float32)

    g = jnp.concatenate([wfx, wfy, wfz], axis=0)
    outT = jnp.dot(w2t_ref[...], g,
                   preferred_element_type=jnp.float32)
    recip = 1.0 / jnp.maximum(cnt, 1.0)
    return outT * recip


def _tc_body(gath_ref, kp_ref, w2t_ref, bias_ref, out_ref):
    for s in range(_TCB // _NPB):
        gt = gath_ref[pl.ds(s * _NPB, _NPB), :].T
        outT = _half(gt, kp_ref, w2t_ref) + bias_ref[...]
        out_ref[pl.ds(s * _NPB, _NPB), :] = outT.T


def kernel(points, neighbor_indices, weights, bias, kernel_points):
    idx32 = neighbor_indices.reshape(-1).astype(jnp.int32)
    idx2d = jnp.pad(idx32, (0, _ROWS_PAD - _ROWS)).reshape(-1, _G)
    tbl4 = jnp.pad(points, ((0, _NPTS - N), (0, 1)))
    tbl16 = jnp.tile(tbl4, (1, 4))
    kp_pad = jnp.concatenate(
        [kernel_points, jnp.full((1, 3), 1e4, jnp.float32)], axis=0)
    w2 = jnp.pad(jnp.transpose(weights, (1, 0, 2)),
                 ((0, 0), (0, 1), (0, 0)))
    w2t = w2.reshape(48, 32).T
    bias2 = bias.reshape(32, 1)

    mesh = plsc.VectorSubcoreMesh(core_axis_name="c", subcore_axis_name="s")
    gathered = pl.kernel(
        _sc_gather,
        out_type=jax.ShapeDtypeStruct((_NPTS, _G), jnp.float32),
        mesh=mesh,
        scratch_types=[
            pltpu.VMEM((_GPC, _G), jnp.int32),
            pltpu.VMEM((_CHR, 16), jnp.float32),
            pltpu.VMEM((_PPC,), jnp.int32),
            pltpu.VMEM((_PPC, 16), jnp.float32),
            pltpu.SemaphoreType.DMA,
            pltpu.VMEM((_GPC, _G), jnp.int32),
            pltpu.VMEM((_CHR, 16), jnp.float32),
            pltpu.VMEM((_PPC,), jnp.int32),
            pltpu.VMEM((_PPC, 16), jnp.float32),
            pltpu.SemaphoreType.DMA,
            pltpu.VMEM((_PPC, _G), jnp.float32),
        ],
        compiler_params=pltpu.CompilerParams(use_tc_tiling_on_sc=False),
    )(tbl16, idx2d)

    grid = (N + _TCB - 1) // _TCB
    out = pl.pallas_call(
        _tc_body,
        grid=(grid,),
        in_specs=[
            pl.BlockSpec((_TCB, _G), lambda i: (i, 0)),
            pl.BlockSpec((_KP, 3), lambda i: (0, 0)),
            pl.BlockSpec((32, 48), lambda i: (0, 0)),
            pl.BlockSpec((32, 1), lambda i: (0, 0)),
        ],
        out_specs=pl.BlockSpec((_TCB, 32), lambda i: (i, 0)),
        out_shape=jax.ShapeDtypeStruct((N, 32), jnp.float32),
    )(gathered, kp_pad, w2t, bias2)
    return out


# speedup vs baseline: 1.3374x; 1.2132x over previous
import jax
import jax.numpy as jnp
from jax import lax
from jax.experimental import pallas as pl
from jax.experimental.pallas import tpu as pltpu
from jax.experimental.pallas import tpu_sc as plsc

N = 50000
H = 16
K = 15
SIGMA = 0.05

_NC = 2
_NS = 16
_NW = _NC * _NS
_ROWS = N * H
_G = 128
_GPW = 196
_ROWS_PAD = _NW * _GPW * _G
_GPC = 14
_NCH = _GPW // _GPC
_CHR = _GPC * _G

_NPTS = _ROWS_PAD // H
_PPW = _NPTS // _NW
_PPC = _CHR // H

_NPB = 128
_TCB = 512
_KP = 16


def _sc_gather(tbl_hbm, idx_hbm, out_hbm,
               idx_a, rows_a, ident_a, qrows_a, sem_a,
               idx_b, rows_b, ident_b, qrows_b, sem_b,
               wide_v):
    wid = lax.axis_index("s") * _NC + lax.axis_index("c")
    iota = lax.iota(jnp.int32, 16)
    m4 = iota < 4
    m8 = iota < 8
    m12 = iota < 12
    gbase = wid * _GPW
    pbase = wid * _PPW
    bufs = [(idx_a, rows_a, ident_a, qrows_a, sem_a),
            (idx_b, rows_b, ident_b, qrows_b, sem_b)]

    def fire(i, buf):
        idx_v, rows_v, ident_v, qrows_v, sem = buf
        pltpu.sync_copy(idx_hbm.at[pl.ds(gbase + i * _GPC, _GPC), :], idx_v)
        p0 = pbase + i * _PPC
        for g in range(_PPC // 16):
            ident_v[pl.ds(g * 16, 16)] = p0 + g * 16 + iota
        for j in range(_GPC):
            pltpu.async_copy(
                tbl_hbm.at[idx_v.at[j]],
                rows_v.at[pl.ds(j * _G, _G), :],
                sem,
            )
        pltpu.async_copy(tbl_hbm.at[ident_v], qrows_v, sem)

    def drain_and_emit(i, buf):
        idx_v, rows_v, ident_v, qrows_v, sem = buf
        for j in range(_GPC):
            pltpu.make_async_copy(
                tbl_hbm.at[idx_v.at[j]],
                rows_v.at[pl.ds(j * _G, _G), :],
                sem,
            ).wait()
        pltpu.make_async_copy(tbl_hbm.at[ident_v], qrows_v, sem).wait()

        def rp(p, c2):
            m0 = p * 16
            for s in range(4):
                v = jnp.where(
                    m4, rows_v[m0 + 4 * s, :],
                    jnp.where(m8, rows_v[m0 + 4 * s + 1, :],
                              jnp.where(m12, rows_v[m0 + 4 * s + 2, :],
                                        rows_v[m0 + 4 * s + 3, :])))
                wide_v[p, pl.ds(s * 16, 16)] = v
            wide_v[p, pl.ds(64, 16)] = qrows_v[p, :]
            return c2

        lax.fori_loop(0, _PPC, rp, 0)
        pltpu.sync_copy(
            wide_v, out_hbm.at[pl.ds(pbase + i * _PPC, _PPC), :])

    fire(0, bufs[0])
    for i in range(_NCH):
        if i + 1 < _NCH:
            fire(i + 1, bufs[(i + 1) % 2])
        drain_and_emit(i, bufs[i % 2])


def _half(gt, kp_ref, w2t_ref):
    kp = kp_ref[...]
    kpx = kp[:, 0:1]
    kpy = kp[:, 1:2]
    kpz = kp[:, 2:3]
    xq = gt[64:65, :]
    yq = gt[65:66, :]
    zq = gt[66:67, :]

    wfx = jnp.zeros((_KP, _NPB), jnp.float32)
    wfy = jnp.zeros((_KP, _NPB), jnp.float32)
    wfz = jnp.zeros((_KP, _NPB), jnp.float32)
    cnt = jnp.zeros((1, _NPB), jnp.float32)
    for h in range(H):
        xh = gt[4 * h:4 * h + 1, :]
        yh = gt[4 * h + 1:4 * h + 2, :]
        zh = gt[4 * h + 2:4 * h + 3, :]
        dx = (xh - xq) - kpx
        dy = (yh - yq) - kpy
        dz = (zh - zq) - kpz
        sq = dx * dx + dy * dy + dz * dz
        w = jnp.maximum(1.0 - jnp.sqrt(sq) * (1.0 / SIGMA), 0.0)
        wfx = wfx + w * xh
        wfy = wfy + w * yh
        wfz = wfz + w * zh
        cnt = cnt + (xh + yh + zh > 0.0).astype(jnp.float32)

    g = jnp.concatenate([wfx, wfy, wfz], axis=0)
    outT = jnp.dot(w2t_ref[...], g,
                   preferred_element_type=jnp.float32)
    recip = 1.0 / jnp.maximum(cnt, 1.0)
    return outT * recip


def _tc_body(gath_ref, kp_ref, w2t_ref, bias_ref, out_ref):
    for s in range(_TCB // _NPB):
        gt = gath_ref[pl.ds(s * _NPB, _NPB), :].T
        outT = _half(gt, kp_ref, w2t_ref) + bias_ref[...]
        out_ref[pl.ds(s * _NPB, _NPB), :] = outT.T


def kernel(points, neighbor_indices, weights, bias, kernel_points):
    idx32 = neighbor_indices.reshape(-1).astype(jnp.int32)
    idx2d = jnp.pad(idx32, (0, _ROWS_PAD - _ROWS)).reshape(-1, _G)
    tbl4 = jnp.pad(points, ((0, _NPTS - N), (0, 1)))
    tbl16 = jnp.tile(tbl4, (1, 4))
    kp_pad = jnp.concatenate(
        [kernel_points, jnp.full((1, 3), 1e4, jnp.float32)], axis=0)
    w2 = jnp.pad(jnp.transpose(weights, (1, 0, 2)),
                 ((0, 0), (0, 1), (0, 0)))
    w2t = w2.reshape(48, 32).T
    bias2 = bias.reshape(32, 1)

    mesh = plsc.VectorSubcoreMesh(core_axis_name="c", subcore_axis_name="s")
    gathered = pl.kernel(
        _sc_gather,
        out_type=jax.ShapeDtypeStruct((_NPTS, _G), jnp.float32),
        mesh=mesh,
        scratch_types=[
            pltpu.VMEM((_GPC, _G), jnp.int32),
            pltpu.VMEM((_CHR, 16), jnp.float32),
            pltpu.VMEM((_PPC,), jnp.int32),
            pltpu.VMEM((_PPC, 16), jnp.float32),
            pltpu.SemaphoreType.DMA,
            pltpu.VMEM((_GPC, _G), jnp.int32),
            pltpu.VMEM((_CHR, 16), jnp.float32),
            pltpu.VMEM((_PPC,), jnp.int32),
            pltpu.VMEM((_PPC, 16), jnp.float32),
            pltpu.SemaphoreType.DMA,
            pltpu.VMEM((_PPC, _G), jnp.float32),
        ],
        compiler_params=pltpu.CompilerParams(use_tc_tiling_on_sc=False),
    )(tbl16, idx2d)

    grid = (N + _TCB - 1) // _TCB
    out = pl.pallas_call(
        _tc_body,
        grid=(grid,),
        in_specs=[
            pl.BlockSpec((_TCB, _G), lambda i: (i, 0)),
            pl.BlockSpec((_KP, 3), lambda i: (0, 0)),
            pl.BlockSpec((32, 48), lambda i: (0, 0)),
            pl.BlockSpec((32, 1), lambda i: (0, 0)),
        ],
        out_specs=pl.BlockSpec((_TCB, 32), lambda i: (i, 0)),
        out_shape=jax.ShapeDtypeStruct((N, 32), jnp.float32),
    )(gathered, kp_pad, w2t, bias2)
    return out


# speedup vs baseline: 1.4263x; 1.0665x over previous
import jax
import jax.numpy as jnp
from jax import lax
from jax.experimental import pallas as pl
from jax.experimental.pallas import tpu as pltpu
from jax.experimental.pallas import tpu_sc as plsc

N = 50000
H = 16
K = 15
SIGMA = 0.05

_NC = 2
_NS = 16
_NW = _NC * _NS
_ROWS = N * H
_G = 128
_GPW = 196
_ROWS_PAD = _NW * _GPW * _G
_GPC = 14
_NCH = _GPW // _GPC
_CHR = _GPC * _G

_NPTS = _ROWS_PAD // H
_PPW = _NPTS // _NW
_PPC = _CHR // H

_NPB = 128
_TCB = 1024
_KP = 16


def _sc_gather(tbl_hbm, idx_hbm, out_hbm,
               idx_a, rows_a, ident_a, qrows_a, sem_a,
               idx_b, rows_b, ident_b, qrows_b, sem_b,
               wide_v):
    wid = lax.axis_index("s") * _NC + lax.axis_index("c")
    iota = lax.iota(jnp.int32, 16)
    m4 = iota < 4
    m8 = iota < 8
    m12 = iota < 12
    gbase = wid * _GPW
    pbase = wid * _PPW
    bufs = [(idx_a, rows_a, ident_a, qrows_a, sem_a),
            (idx_b, rows_b, ident_b, qrows_b, sem_b)]

    def fire(i, buf):
        idx_v, rows_v, ident_v, qrows_v, sem = buf
        pltpu.sync_copy(idx_hbm.at[pl.ds(gbase + i * _GPC, _GPC), :], idx_v)
        p0 = pbase + i * _PPC
        for g in range(_PPC // 16):
            ident_v[pl.ds(g * 16, 16)] = p0 + g * 16 + iota
        for j in range(_GPC):
            pltpu.async_copy(
                tbl_hbm.at[idx_v.at[j]],
                rows_v.at[pl.ds(j * _G, _G), :],
                sem,
            )
        pltpu.async_copy(tbl_hbm.at[ident_v], qrows_v, sem)

    def drain_and_emit(i, buf):
        idx_v, rows_v, ident_v, qrows_v, sem = buf
        for j in range(_GPC):
            pltpu.make_async_copy(
                tbl_hbm.at[idx_v.at[j]],
                rows_v.at[pl.ds(j * _G, _G), :],
                sem,
            ).wait()
        pltpu.make_async_copy(tbl_hbm.at[ident_v], qrows_v, sem).wait()

        def rp(p, c2):
            m0 = p * 16
            for s in range(4):
                v = jnp.where(
                    m4, rows_v[m0 + 4 * s, :],
                    jnp.where(m8, rows_v[m0 + 4 * s + 1, :],
                              jnp.where(m12, rows_v[m0 + 4 * s + 2, :],
                                        rows_v[m0 + 4 * s + 3, :])))
                wide_v[p, pl.ds(s * 16, 16)] = v
            wide_v[p, pl.ds(64, 16)] = qrows_v[p, :]
            return c2

        lax.fori_loop(0, _PPC, rp, 0)
        pltpu.sync_copy(
            wide_v, out_hbm.at[pl.ds(pbase + i * _PPC, _PPC), :])

    fire(0, bufs[0])
    for i in range(_NCH):
        if i + 1 < _NCH:
            fire(i + 1, bufs[(i + 1) % 2])
        drain_and_emit(i, bufs[i % 2])


def _half(gt, kp_ref, w2t_ref):
    kp = kp_ref[...]
    kpx = kp[:, 0:1]
    kpy = kp[:, 1:2]
    kpz = kp[:, 2:3]
    xq = gt[64:65, :]
    yq = gt[65:66, :]
    zq = gt[66:67, :]

    wfx = jnp.zeros((_KP, _NPB), jnp.float32)
    wfy = jnp.zeros((_KP, _NPB), jnp.float32)
    wfz = jnp.zeros((_KP, _NPB), jnp.float32)
    cnt = jnp.zeros((1, _NPB), jnp.float32)
    for h in range(H):
        xh = gt[4 * h:4 * h + 1, :]
        yh = gt[4 * h + 1:4 * h + 2, :]
        zh = gt[4 * h + 2:4 * h + 3, :]
        dx = (xh - xq) - kpx
        dy = (yh - yq) - kpy
        dz = (zh - zq) - kpz
        sq = dx * dx + dy * dy + dz * dz
        w = jnp.maximum(1.0 - jnp.sqrt(sq) * (1.0 / SIGMA), 0.0)
        wfx = wfx + w * xh
        wfy = wfy + w * yh
        wfz = wfz + w * zh
        cnt = cnt + (xh + yh + zh > 0.0).astype(jnp.float32)

    g = jnp.concatenate([wfx, wfy, wfz], axis=0)
    outT = jnp.dot(w2t_ref[...], g,
                   preferred_element_type=jnp.float32)
    recip = 1.0 / jnp.maximum(cnt, 1.0)
    return outT * recip


def _tc_body(gath_ref, kp_ref, w2t_ref, bias_ref, out_ref):
    for s in range(_TCB // _NPB):
        gt = gath_ref[pl.ds(s * _NPB, _NPB), :].T
        outT = _half(gt, kp_ref, w2t_ref) + bias_ref[...]
        out_ref[pl.ds(s * _NPB, _NPB), :] = outT.T


def kernel(points, neighbor_indices, weights, bias, kernel_points):
    idx32 = neighbor_indices.reshape(-1).astype(jnp.int32)
    idx2d = jnp.pad(idx32, (0, _ROWS_PAD - _ROWS)).reshape(-1, _G)
    tbl4 = jnp.pad(points, ((0, _NPTS - N), (0, 1)))
    tbl16 = jnp.tile(tbl4, (1, 4))
    kp_pad = jnp.concatenate(
        [kernel_points, jnp.full((1, 3), 1e4, jnp.float32)], axis=0)
    w2 = jnp.pad(jnp.transpose(weights, (1, 0, 2)),
                 ((0, 0), (0, 1), (0, 0)))
    w2t = w2.reshape(48, 32).T
    bias2 = bias.reshape(32, 1)

    mesh = plsc.VectorSubcoreMesh(core_axis_name="c", subcore_axis_name="s")
    gathered = pl.kernel(
        _sc_gather,
        out_type=jax.ShapeDtypeStruct((_NPTS, _G), jnp.float32),
        mesh=mesh,
        scratch_types=[
            pltpu.VMEM((_GPC, _G), jnp.int32),
            pltpu.VMEM((_CHR, 16), jnp.float32),
            pltpu.VMEM((_PPC,), jnp.int32),
            pltpu.VMEM((_PPC, 16), jnp.float32),
            pltpu.SemaphoreType.DMA,
            pltpu.VMEM((_GPC, _G), jnp.int32),
            pltpu.VMEM((_CHR, 16), jnp.float32),
            pltpu.VMEM((_PPC,), jnp.int32),
            pltpu.VMEM((_PPC, 16), jnp.float32),
            pltpu.SemaphoreType.DMA,
            pltpu.VMEM((_PPC, _G), jnp.float32),
        ],
        compiler_params=pltpu.CompilerParams(use_tc_tiling_on_sc=False),
    )(tbl16, idx2d)

    grid = (N + _TCB - 1) // _TCB
    out = pl.pallas_call(
        _tc_body,
        grid=(grid,),
        in_specs=[
            pl.BlockSpec((_TCB, _G), lambda i: (i, 0)),
            pl.BlockSpec((_KP, 3), lambda i: (0, 0)),
            pl.BlockSpec((32, 48), lambda i: (0, 0)),
            pl.BlockSpec((32, 1), lambda i: (0, 0)),
        ],
        out_specs=pl.BlockSpec((_TCB, 32), lambda i: (i, 0)),
        out_shape=jax.ShapeDtypeStruct((N, 32), jnp.float32),
    )(gathered, kp_pad, w2t, bias2)
    return out
